# baseline (speedup 1.0000x reference)
import jax
import jax.numpy as jnp
from jax import lax
from jax.experimental import pallas as pl
from jax.experimental.pallas import tpu as pltpu
from jax.experimental.pallas import tpu_sc as plsc

N_P = 5000
NODE_SIZE = 10000
FT = 256
HID = 256
OUT = 256
E = 80000

NC, NS, L = 2, 16, 16
SEG = FT // L
RT = 320
ACC_ROWS = NS * RT
EC = 1024
EPAD = 81920
CAP = 128
FLUSH_AT = CAP - L


def _spmm_body(rows_hbm, cols_hbm, w_hbm, table_hbm, zeros_hbm, out_hbm,
               rbuf, cbuf, wbuf, cstage, wstage, rstage, xbuf, acc, sem):
  c = lax.axis_index("c")
  s = lax.axis_index("s")
  lo = s * RT

  pltpu.sync_copy(zeros_hbm, acc)
  zero_i = jnp.zeros((L,), jnp.int32)
  zero_f = jnp.zeros((L,), jnp.float32)
  for g in range(CAP // L):
    cstage[pl.ds(g * L, L)] = zero_i
    rstage[pl.ds(g * L, L)] = zero_i
    wstage[pl.ds(g * L, L)] = zero_f

  def flush():
    pltpu.async_copy(table_hbm.at[cstage], xbuf, sem).wait()

    def fgroup(g, carry):
      w16 = wstage[pl.ds(g * L, L)]
      r16 = rstage[pl.ds(g * L, L)]
      for l in range(L):
        w = w16[l]
        r = r16[l]
        for si in range(SEG):
          sl = pl.ds(si * L, L)
          plsc.addupdate(acc.at[r, sl], xbuf[g * L + l, sl] * w)
      wstage[pl.ds(g * L, L)] = zero_f
      return carry

    lax.fori_loop(0, CAP // L, fgroup, 0)

  def chunk(ch, cnt):
    base = ch * EC
    pltpu.sync_copy(rows_hbm.at[c, pl.ds(base, EC)], rbuf)
    pltpu.sync_copy(cols_hbm.at[c, pl.ds(base, EC)], cbuf)
    pltpu.sync_copy(w_hbm.at[c, pl.ds(base, EC)], wbuf)

    def group(g, cnt2):
      row16 = rbuf[pl.ds(g * L, L)]
      m = (row16 >= lo) & (row16 < lo + RT)
      inc = plsc.cumsum(jnp.where(m, 1, 0))
      pos = cnt2 + inc - 1
      plsc.store_scatter(cstage, [pos], cbuf[pl.ds(g * L, L)], mask=m)
      plsc.store_scatter(wstage, [pos], wbuf[pl.ds(g * L, L)], mask=m)
      plsc.store_scatter(rstage, [pos], row16 - lo, mask=m)
      cnt2 = cnt2 + inc[L - 1]

      @pl.when(cnt2 >= FLUSH_AT)
      def _():
        flush()

      return jnp.where(cnt2 >= FLUSH_AT, 0, cnt2)

    return lax.fori_loop(0, EC // L, group, cnt)

  cnt = lax.fori_loop(0, EPAD // EC, chunk, jnp.int32(0))

  @pl.when(cnt > 0)
  def _():
    flush()

  pltpu.sync_copy(acc, out_hbm.at[c, pl.ds(lo, RT)])


@jax.jit
def _spmm2(table, rows2, cols2, w2, zeros):
  mesh = plsc.VectorSubcoreMesh(core_axis_name="c", subcore_axis_name="s")
  return pl.kernel(
      _spmm_body,
      out_type=jax.ShapeDtypeStruct((NC, ACC_ROWS, FT), jnp.float32),
      mesh=mesh,
      compiler_params=pltpu.CompilerParams(needs_layout_passes=False),
      scratch_types=[
          pltpu.VMEM((EC,), jnp.int32),
          pltpu.VMEM((EC,), jnp.int32),
          pltpu.VMEM((EC,), jnp.float32),
          pltpu.VMEM((CAP,), jnp.int32),
          pltpu.VMEM((CAP,), jnp.float32),
          pltpu.VMEM((CAP,), jnp.int32),
          pltpu.VMEM((CAP, FT), jnp.float32),
          pltpu.VMEM((RT, FT), jnp.float32),
          pltpu.SemaphoreType.DMA,
      ],
  )(rows2, cols2, w2, table, zeros)


def _gcn_matmul_body(mn_ref, w_ref, out_ref):
  out_ref[0] = jnp.maximum(
      jnp.dot(mn_ref[0], w_ref[0], preferred_element_type=jnp.float32), 0.0)


BR = 1280


@jax.jit
def _gcn_layer(mn, w_stack):
  return pl.pallas_call(
      _gcn_matmul_body,
      grid=(NC, ACC_ROWS // BR),
      in_specs=[
          pl.BlockSpec((1, BR, FT), lambda c, i: (c, i, 0)),
          pl.BlockSpec((1, FT, HID), lambda c, i: (c, 0, 0)),
      ],
      out_specs=pl.BlockSpec((1, BR, HID), lambda c, i: (c, i, 0)),
      out_shape=jax.ShapeDtypeStruct((NC, ACC_ROWS, HID), jnp.float32),
  )(mn, w_stack)


def _final_body(mn2_ref, feat_ref, w1_ref, wfa_ref, wfb_ref, b_ref, out_ref):
  v = jnp.maximum(
      jnp.dot(mn2_ref[0], w1_ref[0], preferred_element_type=jnp.float32), 0.0)
  out_ref[...] = (
      jnp.dot(v, wfa_ref[0], preferred_element_type=jnp.float32)
      + jnp.dot(feat_ref[...], wfb_ref[0], preferred_element_type=jnp.float32)
      + b_ref[0])


FR = 1000


@jax.jit
def _final_layer(mn2, features, w1_stack, wfa_stack, wfb_stack, b_stack):
  nb = N_P // FR
  return pl.pallas_call(
      _final_body,
      grid=(NC * nb,),
      in_specs=[
          pl.BlockSpec((1, FR, HID), lambda i: (i // nb, i % nb, 0)),
          pl.BlockSpec((FR, FT), lambda i: (i, 0)),
          pl.BlockSpec((1, HID, HID), lambda i: (i // nb, 0, 0)),
          pl.BlockSpec((1, HID, OUT), lambda i: (i // nb, 0, 0)),
          pl.BlockSpec((1, FT, OUT), lambda i: (i // nb, 0, 0)),
          pl.BlockSpec((1, 1, OUT), lambda i: (i // nb, 0, 0)),
      ],
      out_specs=pl.BlockSpec((FR, OUT), lambda i: (i, 0)),
      out_shape=jax.ShapeDtypeStruct((NODE_SIZE, OUT), jnp.float32),
  )(mn2, features, w1_stack, wfa_stack, wfb_stack, b_stack)


def kernel(features, edge_index_p, edge_weight_p, edge_index_a, edge_weight_a,
           idx_p, idx_a, W0_pa, W0_ap, W1_pa, W1_ap, Wfc_p, bfc_p, Wfc_a,
           bfc_a):
  pad = EPAD - E
  rows2 = jnp.stack([
      jnp.pad(edge_index_p[0], (0, pad)),
      jnp.pad(edge_index_a[0], (0, pad)),
  ])
  cols_l0 = jnp.stack([
      jnp.pad(edge_index_p[1] + N_P, (0, pad)),
      jnp.pad(edge_index_a[1], (0, pad)),
  ])
  cols_l1 = jnp.stack([
      jnp.pad(edge_index_p[1] + ACC_ROWS, (0, pad)),
      jnp.pad(edge_index_a[1], (0, pad)),
  ])
  w2 = jnp.stack([
      jnp.pad(edge_weight_p, (0, pad)),
      jnp.pad(edge_weight_a, (0, pad)),
  ])
  zeros = jnp.zeros((RT, FT), jnp.float32)

  mn = _spmm2(features, rows2, cols_l0, w2, zeros)
  w0_stack = jnp.stack([W0_pa, W0_ap])
  embs1 = _gcn_layer(mn, w0_stack)
  mn2 = _spmm2(embs1.reshape(NC * ACC_ROWS, HID), rows2, cols_l1, w2, zeros)
  w1_stack = jnp.stack([W1_pa, W1_ap])
  wfa_stack = jnp.stack([Wfc_p[:HID], Wfc_a[:HID]])
  wfb_stack = jnp.stack([Wfc_p[HID:], Wfc_a[HID:]])
  b_stack = jnp.stack([bfc_p, bfc_a]).reshape(NC, 1, OUT)
  return _final_layer(mn2, features, w1_stack, wfa_stack, wfb_stack, b_stack)
